# baseline (speedup 1.0000x reference)
import functools

import jax
import jax.numpy as jnp
from jax import lax
from jax.experimental import pallas as pl
from jax.experimental.pallas import tpu as pltpu
from jax.experimental.pallas import tpu_sc as plsc

L = 16
BW = 128
SC = 4
NC, NS = 2, 16


@functools.lru_cache(maxsize=None)
def _build(batch: int, seq: int, dim: int):
    nw = NC * NS
    assert batch == nw * BW * (batch // (nw * BW))
    bpw = batch // nw
    assert bpw == BW and dim == 2 * L and seq % SC == 0
    nchunks = seq // SC
    assert nchunks % 2 == 0
    rows_per_chunk = SC * BW
    trows = SC * dim

    mesh = plsc.VectorSubcoreMesh(core_axis_name="c", subcore_axis_name="s",
                                  num_cores=NC, num_subcores=NS)
    scratch = (
        [pltpu.VMEM((seq, dim), jnp.float32)]
        + [pltpu.VMEM((SC, BW), jnp.int32) for _ in range(2)]
        + [pltpu.VMEM((rows_per_chunk, dim), jnp.float32) for _ in range(2)]
        + [pltpu.VMEM((trows, BW + 1), jnp.float32) for _ in range(2)]
        + [pltpu.SemaphoreType.DMA for _ in range(4)]
    )

    @functools.partial(
        pl.kernel,
        out_type=jax.ShapeDtypeStruct((seq * dim, batch), jnp.float32),
        mesh=mesh,
        scratch_types=scratch,
        compiler_params=pltpu.CompilerParams(use_tc_tiling_on_sc=False,
                                             needs_layout_passes=False),
    )
    def kern(idx_hbm, tok_hbm, pos_hbm, out_hbm, pos_v,
             ic0, ic1, buf0, buf1, tb0, tb1, g0, g1, o0, o1):
        idxc = (ic0, ic1)
        buf = (buf0, buf1)
        tbuf = (tb0, tb1)
        gsem = (g0, g1)
        osem = (o0, o1)

        wid = lax.axis_index("s") * NC + lax.axis_index("c")
        col0 = wid * BW

        pltpu.sync_copy(pos_hbm, pos_v)

        def issue(g, b):
            pltpu.sync_copy(
                idx_hbm.at[pl.ds(g * SC, SC), pl.ds(col0, BW)], idxc[b])
            for si in range(SC):
                pltpu.async_copy(
                    tok_hbm.at[idxc[b].at[si]],
                    buf[b].at[pl.ds(si * BW, BW)], gsem[b])

        def gather_wait(g, b):
            for si in range(SC):
                pltpu.make_async_copy(
                    tok_hbm.at[idxc[b].at[si]],
                    buf[b].at[pl.ds(si * BW, BW)], gsem[b]).wait()

        def out_wait(g, b):
            pltpu.make_async_copy(
                tbuf[b].at[:, pl.ds(0, BW)],
                out_hbm.at[pl.ds(g * trows, trows),
                           pl.ds(col0, BW)], osem[b]).wait()

        iota = lax.iota(jnp.int32, L)
        rowvecs = [iota + j * L for j in range(BW // L)]

        def transpose_add(g, b):
            for si in range(SC):
                s = g * SC + si
                p0 = pos_v[s, pl.ds(0, L)]
                p1 = pos_v[s, pl.ds(L, L)]
                row0 = iota + si * dim
                row1 = row0 + L

                @plsc.parallel_loop(0, BW, unroll=8)
                def scb(bi, si=si, p0=p0, p1=p1, row0=row0, row1=row1, b=b):
                    r = si * BW + bi
                    col = jnp.full((L,), bi, jnp.int32)
                    v0 = buf[b][r, pl.ds(0, L)] + p0
                    v1 = buf[b][r, pl.ds(L, L)] + p1
                    plsc.store_scatter(tbuf[b], [row0, col], v0)
                    plsc.store_scatter(tbuf[b], [row1, col], v1)

        issue(0, 0)
        issue(1, 1)

        def chunk_pair(i, _):
            for b in range(2):
                g = i * 2 + b
                gather_wait(g, b)

                @pl.when(g >= 2)
                def _(g=g, b=b):
                    out_wait(g - 2, b)

                transpose_add(g, b)
                pltpu.async_copy(
                    tbuf[b].at[:, pl.ds(0, BW)],
                    out_hbm.at[pl.ds(g * trows, trows),
                               pl.ds(col0, BW)], osem[b])

                @pl.when(g + 2 < nchunks)
                def _(g=g, b=b):
                    issue(g + 2, b)
            return 0

        lax.fori_loop(0, nchunks // 2, chunk_pair, 0)
        out_wait(nchunks - 2, 0)
        out_wait(nchunks - 1, 1)

    return kern


def kernel(inputs, token_table, position_table):
    batch, seq = inputs.shape
    dim = token_table.shape[1]
    idx_t = inputs.astype(jnp.int32).T
    out = _build(batch, seq, dim)(
        idx_t, token_table.astype(jnp.float32),
        position_table.astype(jnp.float32))
    return out.reshape(seq, dim, batch).transpose(2, 0, 1)


# speedup vs baseline: 1.0028x; 1.0028x over previous
import functools

import jax
import jax.numpy as jnp
from jax import lax
from jax.experimental import pallas as pl
from jax.experimental.pallas import tpu as pltpu
from jax.experimental.pallas import tpu_sc as plsc

L = 16
BW = 128
SC = 4
NC, NS = 2, 16


@functools.lru_cache(maxsize=None)
def _build(batch: int, seq: int, dim: int):
    nw = NC * NS
    assert batch == nw * BW * (batch // (nw * BW))
    bpw = batch // nw
    assert bpw == BW and dim == 2 * L and seq % SC == 0
    nchunks = seq // SC
    assert nchunks % 2 == 0
    rows_per_chunk = SC * BW
    trows = SC * dim

    mesh = plsc.VectorSubcoreMesh(core_axis_name="c", subcore_axis_name="s",
                                  num_cores=NC, num_subcores=NS)
    scratch = (
        [pltpu.VMEM((seq, dim), jnp.float32)]
        + [pltpu.VMEM((SC, BW), jnp.int32) for _ in range(2)]
        + [pltpu.VMEM((rows_per_chunk, dim), jnp.float32) for _ in range(2)]
        + [pltpu.VMEM((trows, BW + 1), jnp.float32) for _ in range(2)]
        + [pltpu.SemaphoreType.DMA for _ in range(4)]
    )

    @functools.partial(
        pl.kernel,
        out_type=jax.ShapeDtypeStruct((seq * dim, batch), jnp.float32),
        mesh=mesh,
        scratch_types=scratch,
        compiler_params=pltpu.CompilerParams(use_tc_tiling_on_sc=False,
                                             needs_layout_passes=False),
    )
    def kern(idx_hbm, tok_hbm, pos_hbm, out_hbm, pos_v,
             ic0, ic1, buf0, buf1, tb0, tb1, g0, g1, o0, o1):
        idxc = (ic0, ic1)
        buf = (buf0, buf1)
        tbuf = (tb0, tb1)
        gsem = (g0, g1)
        osem = (o0, o1)

        wid = lax.axis_index("s") * NC + lax.axis_index("c")
        col0 = wid * BW

        pltpu.sync_copy(pos_hbm, pos_v)

        def issue(g, b):
            pltpu.sync_copy(
                idx_hbm.at[pl.ds(g * SC, SC), pl.ds(col0, BW)], idxc[b])
            for si in range(SC):
                pltpu.async_copy(
                    tok_hbm.at[idxc[b].at[si]],
                    buf[b].at[pl.ds(si * BW, BW)], gsem[b])

        def gather_wait(g, b):
            for si in range(SC):
                pltpu.make_async_copy(
                    tok_hbm.at[idxc[b].at[si]],
                    buf[b].at[pl.ds(si * BW, BW)], gsem[b]).wait()

        def out_wait(g, b):
            pltpu.make_async_copy(
                tbuf[b].at[:, pl.ds(0, BW)],
                out_hbm.at[pl.ds(g * trows, trows),
                           pl.ds(col0, BW)], osem[b]).wait()

        iota = lax.iota(jnp.int32, L)
        rowvecs = [iota + j * L for j in range(BW // L)]

        def transpose_add(g, b):
            for si in range(SC):
                s = g * SC + si
                p0 = pos_v[s, pl.ds(0, L)]
                p1 = pos_v[s, pl.ds(L, L)]
                row0 = iota + si * dim
                row1 = row0 + L

                @plsc.parallel_loop(0, BW, unroll=4)
                def scb(bi, si=si, p0=p0, p1=p1, row0=row0, row1=row1, b=b):
                    r = si * BW + bi
                    col = jnp.full((L,), bi, jnp.int32)
                    v0 = buf[b][r, pl.ds(0, L)] + p0
                    v1 = buf[b][r, pl.ds(L, L)] + p1
                    plsc.store_scatter(tbuf[b], [row0, col], v0)
                    plsc.store_scatter(tbuf[b], [row1, col], v1)

        issue(0, 0)
        issue(1, 1)

        def chunk_pair(i, _):
            for b in range(2):
                g = i * 2 + b
                gather_wait(g, b)

                @pl.when(g >= 2)
                def _(g=g, b=b):
                    out_wait(g - 2, b)

                transpose_add(g, b)
                pltpu.async_copy(
                    tbuf[b].at[:, pl.ds(0, BW)],
                    out_hbm.at[pl.ds(g * trows, trows),
                               pl.ds(col0, BW)], osem[b])

                @pl.when(g + 2 < nchunks)
                def _(g=g, b=b):
                    issue(g + 2, b)
            return 0

        lax.fori_loop(0, nchunks // 2, chunk_pair, 0)
        out_wait(nchunks - 2, 0)
        out_wait(nchunks - 1, 1)

    return kern


def kernel(inputs, token_table, position_table):
    batch, seq = inputs.shape
    dim = token_table.shape[1]
    idx_t = inputs.astype(jnp.int32).T
    out = _build(batch, seq, dim)(
        idx_t, token_table.astype(jnp.float32),
        position_table.astype(jnp.float32))
    return out.reshape(seq, dim, batch).transpose(2, 0, 1)
